# baseline (speedup 1.0000x reference)
import functools

import jax
import jax.numpy as jnp
from jax import lax
from jax.experimental import pallas as pl
from jax.experimental.pallas import tpu as pltpu
from jax.experimental.pallas import tpu_sc as plsc

B = 16384
C = 1000
CP = 1024
F = 128
TAU = 2.0
ETA = 0.5
WEIGHT_CENT = 0.003
BB = 1024
NB = B // BB

NC = 2
NS = 16
NW = NC * NS
PER_W = 2 * B // NW
CHUNK = 128
NCHUNK = PER_W // CHUNK

_HIGH = jax.lax.Precision.HIGHEST


def _stage_a_body(m_ref, mn_ref, k_ref, h_ref, mnorm2_ref):
    m = m_ref[...]
    ssq = jnp.sum(m * m, axis=1, keepdims=True)
    inv = jax.lax.rsqrt(jnp.maximum(ssq, 1e-30))
    mn = m * inv
    gwm = jax.lax.dot_general(mn, mn, (((1,), (1,)), ((), ())),
                              precision=_HIGH,
                              preferred_element_type=jnp.float32)
    colmask = jax.lax.broadcasted_iota(jnp.int32, (1, CP), 1) < C
    lg = gwm * (1.0 / TAU)
    lgm = jnp.where(colmask, lg, -1e30)
    mx = jnp.max(lgm, axis=1, keepdims=True)
    e = jnp.where(colmask, jnp.exp(lg - mx), 0.0)
    z = jnp.sum(e, axis=1, keepdims=True)
    p = e / z
    logp = jnp.where(colmask, (lg - mx) - jnp.log(z), 0.0)
    h_ref[...] = jnp.sum(p * logp, axis=1, keepdims=True)
    k_ref[...] = jax.lax.dot_general(p, mn, (((1,), (0,)), ((), ())),
                                     precision=_HIGH,
                                     preferred_element_type=jnp.float32)
    mn_ref[...] = mn * (1.0 / TAU)
    mnorm2_ref[...] = ssq


def _scale_body(xf_ref, lam_ref, w_ref, wn_ref, lam16_ref):
    xf = xf_ref[...]
    lam = lam_ref[...]
    ssq = jnp.sum(xf * xf, axis=1, keepdims=True)
    w_ref[...] = lam * xf
    wn_ref[...] = (lam * jax.lax.rsqrt(ssq)) * xf
    lane0 = jax.lax.broadcasted_iota(jnp.int32, (1, F), 1) == 0
    lam16_ref[...] = jnp.where(lane0, lam, 0.0)


def _sc_scatter_body(w_hbm, wn_hbm, idx_hbm, lam16_hbm, zeros_hbm,
                     g_out, gn_out, cnt_out,
                     g_sp, gn_sp, cnt_sp,
                     wv, wnv, idxv, lamv, rb, in_sems, out_sem):
    cid = lax.axis_index("c")
    sid = lax.axis_index("s")

    @pl.when(sid == 0)
    def _():
        pltpu.sync_copy(zeros_hbm, g_sp)
        pltpu.sync_copy(zeros_hbm, gn_sp)
        pltpu.sync_copy(zeros_hbm, cnt_sp)

    plsc.subcore_barrier()

    wid = cid * NS + sid
    base_w = wid * PER_W

    def _fill(k, b):
        base = base_w + k * CHUNK
        pltpu.async_copy(idx_hbm.at[pl.ds(base, CHUNK)], idxv.at[b],
                         in_sems.at[b])
        pltpu.async_copy(lam16_hbm.at[pl.ds(base, CHUNK)], lamv.at[b],
                         in_sems.at[b])
        pltpu.async_copy(w_hbm.at[pl.ds(base, CHUNK)], wv.at[b],
                         in_sems.at[b])
        pltpu.async_copy(wn_hbm.at[pl.ds(base, CHUNK)], wnv.at[b],
                         in_sems.at[b])

    def _drain(b):
        pltpu.make_async_copy(idx_hbm.at[pl.ds(0, CHUNK)], idxv.at[b],
                              in_sems.at[b]).wait()
        pltpu.make_async_copy(lam16_hbm.at[pl.ds(0, CHUNK)], lamv.at[b],
                              in_sems.at[b]).wait()
        pltpu.make_async_copy(w_hbm.at[pl.ds(0, CHUNK)], wv.at[b],
                              in_sems.at[b]).wait()
        pltpu.make_async_copy(wn_hbm.at[pl.ds(0, CHUNK)], wnv.at[b],
                              in_sems.at[b]).wait()

    _fill(0, 0)
    for k in range(NCHUNK):
        b = k % 2
        if k + 1 < NCHUNK:
            _fill(k + 1, 1 - b)
        _drain(b)
        pltpu.sync_copy(wv.at[b], g_sp.at[idxv.at[b]], add=True)
        pltpu.sync_copy(wnv.at[b], gn_sp.at[idxv.at[b]], add=True)
        pltpu.sync_copy(lamv.at[b], cnt_sp.at[idxv.at[b]], add=True)

    pltpu.sync_copy(g_sp.at[pl.ds(0, 8)], rb)
    pltpu.sync_copy(gn_sp.at[pl.ds(0, 8)], rb)
    pltpu.sync_copy(cnt_sp.at[pl.ds(0, 8)], rb)
    plsc.subcore_barrier()

    @pl.when(sid == 0)
    def _():
        pltpu.sync_copy(g_sp, g_out.at[cid])
        pltpu.sync_copy(gn_sp, gn_out.at[cid])
        pltpu.sync_copy(cnt_sp, cnt_out.at[cid])


def _stage_b_body(xf_ref, lac_ref, lbc_ref, mn_ref, slse_ref, sxx_ref):
    i = pl.program_id(0)

    @pl.when(i == 0)
    def _():
        slse_ref[...] = jnp.zeros_like(slse_ref)
        sxx_ref[...] = jnp.zeros_like(sxx_ref)

    xf = xf_ref[...]
    xssq = jnp.sum(xf * xf, axis=1, keepdims=True)
    xfn = xf * jax.lax.rsqrt(xssq)
    mn = mn_ref[...]
    st = jax.lax.dot_general(xfn, mn, (((1,), (1,)), ((), ())),
                             precision=_HIGH,
                             preferred_element_type=jnp.float32)
    z = jnp.sum(jnp.exp(st), axis=1, keepdims=True)
    lse = jnp.log(z - float(CP - C))

    lam2c = lac_ref[...] + lbc_ref[...]
    slse_ref[...] += jnp.sum(lam2c * lse)
    sxx_ref[...] += jnp.sum(lam2c * xssq)


def _stage_c_body(gp_ref, gnp_ref, cntp_ref, h_ref, mnorm2_ref, k_ref,
                  m_ref, ginit_ref, cinit_ref, slse_ref, sxx_ref,
                  loss_ref, gout_ref, cntout_ref):
    gs = gp_ref[0] + gp_ref[1]
    gns = gnp_ref[0] + gnp_ref[1]
    cs = cntp_ref[:CP] + cntp_ref[CP:]
    center = (sxx_ref[0, 0]
              - 2.0 * jnp.sum(gs * m_ref[...])
              + jnp.sum(cs * mnorm2_ref[...])) * (WEIGHT_CENT / B)
    sim = (jnp.sum(cs * h_ref[...])
           + slse_ref[0, 0]
           - jnp.sum(gns * k_ref[...]) * (1.0 / TAU)) * (ETA / B)
    loss_ref[...] = jnp.full((1, 1), sim + center, jnp.float32)
    gout_ref[...] = gs + ginit_ref[...]
    cntout_ref[...] = cs + cinit_ref[...]


@jax.jit
def kernel(xf, target_a, target_b, lam_a, lam_b, epoch, matrix, grad_buf,
           count_buf):
    f32 = jnp.float32
    m_pad = jnp.pad(matrix, ((0, CP - C), (0, 0)))
    g_init = jnp.pad(grad_buf, ((0, CP - C), (0, 0)))
    c_init = jnp.pad(count_buf, ((0, CP - C), (0, 0)))

    mn, k, h, mnorm2 = pl.pallas_call(
        _stage_a_body,
        out_shape=[
            jax.ShapeDtypeStruct((CP, F), f32),
            jax.ShapeDtypeStruct((CP, F), f32),
            jax.ShapeDtypeStruct((CP, 1), f32),
            jax.ShapeDtypeStruct((CP, 1), f32),
        ],
    )(m_pad)

    lam_cat = jnp.concatenate([lam_a, lam_b]).reshape(2 * B, 1)
    idx_cat = jnp.concatenate([target_a, target_b]).astype(jnp.int32)
    w, wn, lam16 = pl.pallas_call(
        _scale_body,
        grid=(2 * NB,),
        in_specs=[
            pl.BlockSpec((BB, F), lambda i: (i % NB, 0)),
            pl.BlockSpec((BB, 1), lambda i: (i, 0)),
        ],
        out_specs=[pl.BlockSpec((BB, F), lambda i: (i, 0)),
                   pl.BlockSpec((BB, F), lambda i: (i, 0)),
                   pl.BlockSpec((BB, F), lambda i: (i, 0))],
        out_shape=[
            jax.ShapeDtypeStruct((2 * B, F), f32),
            jax.ShapeDtypeStruct((2 * B, F), f32),
            jax.ShapeDtypeStruct((2 * B, F), f32),
        ],
    )(xf, lam_cat)

    zeros_cpf = jnp.zeros((CP, F), f32)
    sc_scatter = pl.kernel(
        _sc_scatter_body,
        out_type=[
            jax.ShapeDtypeStruct((NC, CP, F), f32),
            jax.ShapeDtypeStruct((NC, CP, F), f32),
            jax.ShapeDtypeStruct((NC, CP, F), f32),
        ],
        mesh=plsc.VectorSubcoreMesh(core_axis_name="c", subcore_axis_name="s"),
        scratch_types=[
            pltpu.VMEM_SHARED((CP, F), f32),
            pltpu.VMEM_SHARED((CP, F), f32),
            pltpu.VMEM_SHARED((CP, F), f32),
            pltpu.VMEM((2, CHUNK, F), f32),
            pltpu.VMEM((2, CHUNK, F), f32),
            pltpu.VMEM((2, CHUNK), jnp.int32),
            pltpu.VMEM((2, CHUNK, F), f32),
            pltpu.VMEM((8, F), f32),
            pltpu.SemaphoreType.DMA((2,)),
            pltpu.SemaphoreType.DMA,
        ],
    )
    g_parts, gn_parts, cnt_parts = sc_scatter(
        w, wn, idx_cat, lam16, zeros_cpf)

    la_c = lam_a.reshape(B, 1)
    lb_c = lam_b.reshape(B, 1)
    full = lambda r, c: pl.BlockSpec((r, c), lambda i: (0, 0))
    slse, sxx = pl.pallas_call(
        _stage_b_body,
        grid=(NB,),
        in_specs=[
            pl.BlockSpec((BB, F), lambda i: (i, 0)),
            pl.BlockSpec((BB, 1), lambda i: (i, 0)),
            pl.BlockSpec((BB, 1), lambda i: (i, 0)),
            full(CP, F),
        ],
        out_specs=[full(1, 1), full(1, 1)],
        out_shape=[
            jax.ShapeDtypeStruct((1, 1), f32),
            jax.ShapeDtypeStruct((1, 1), f32),
        ],
    )(xf, la_c, lb_c, mn)

    cnt_packed = cnt_parts[:, :, :1].reshape(NC * CP, 1)
    loss, g_out, cnt_out = pl.pallas_call(
        _stage_c_body,
        out_shape=[
            jax.ShapeDtypeStruct((1, 1), f32),
            jax.ShapeDtypeStruct((CP, F), f32),
            jax.ShapeDtypeStruct((CP, 1), f32),
        ],
    )(g_parts, gn_parts, cnt_packed, h, mnorm2, k, m_pad, g_init,
      c_init, slse, sxx)

    loss = jnp.where(epoch >= 0, loss[0, 0], jnp.nan)
    return (loss, g_out[:C], cnt_out[:C])


# speedup vs baseline: 1.2785x; 1.2785x over previous
import functools

import jax
import jax.numpy as jnp
from jax import lax
from jax.experimental import pallas as pl
from jax.experimental.pallas import tpu as pltpu
from jax.experimental.pallas import tpu_sc as plsc

B = 16384
C = 1000
CP = 1024
F = 128
TAU = 2.0
ETA = 0.5
WEIGHT_CENT = 0.003
BB = 1024
NB = B // BB

NC = 2
NS = 16
NW = NC * NS
PER_W = 2 * B // NW
CHUNK = 128
NCHUNK = PER_W // CHUNK

_HIGH = jax.lax.Precision.HIGHEST


def _stage_a_body(m_ref, mn_ref, k_ref, h_ref, mnorm2_ref):
    m = m_ref[...]
    ssq = jnp.sum(m * m, axis=1, keepdims=True)
    inv = jax.lax.rsqrt(jnp.maximum(ssq, 1e-30))
    mn = m * inv
    gwm = jax.lax.dot_general(mn, mn, (((1,), (1,)), ((), ())),
                              precision=_HIGH,
                              preferred_element_type=jnp.float32)
    lg = gwm * (1.0 / TAU)
    e = jnp.exp(lg)
    z = jnp.sum(e, axis=1, keepdims=True) - float(CP - C)
    zinv = 1.0 / z
    logz = jnp.log(z)
    p = e * zinv
    h_all = jnp.sum(p * (lg - logz), axis=1, keepdims=True)
    h_ref[...] = h_all + float(CP - C) * logz * zinv
    k_ref[...] = jax.lax.dot_general(p, mn, (((1,), (0,)), ((), ())),
                                     precision=_HIGH,
                                     preferred_element_type=jnp.float32)
    mn_ref[...] = mn * (1.0 / TAU)
    mnorm2_ref[...] = ssq


def _scale_body(xf_ref, lam_ref, w_ref, wn_ref, lam16_ref):
    xf = xf_ref[...]
    lam = lam_ref[...]
    ssq = jnp.sum(xf * xf, axis=1, keepdims=True)
    w_ref[...] = lam * xf
    wn_ref[...] = (lam * jax.lax.rsqrt(ssq)) * xf
    lane0 = jax.lax.broadcasted_iota(jnp.int32, (1, F), 1) == 0
    lam16_ref[...] = jnp.where(lane0, lam, 0.0)


def _sc_scatter_body(w_hbm, wn_hbm, idx_hbm, lam16_hbm, zeros_hbm,
                     g_out, gn_out, cnt_out,
                     g_sp, gn_sp, cnt_sp,
                     wv, wnv, idxv, lamv, rb, in_sems, out_sem):
    cid = lax.axis_index("c")
    sid = lax.axis_index("s")

    @pl.when(sid == 0)
    def _():
        pltpu.sync_copy(zeros_hbm, g_sp)
        pltpu.sync_copy(zeros_hbm, gn_sp)
        pltpu.sync_copy(zeros_hbm, cnt_sp)

    plsc.subcore_barrier()

    wid = cid * NS + sid
    base_w = wid * PER_W

    def _fill(k, b):
        base = base_w + k * CHUNK
        pltpu.async_copy(idx_hbm.at[pl.ds(base, CHUNK)], idxv.at[b],
                         in_sems.at[b])
        pltpu.async_copy(lam16_hbm.at[pl.ds(base, CHUNK)], lamv.at[b],
                         in_sems.at[b])
        pltpu.async_copy(w_hbm.at[pl.ds(base, CHUNK)], wv.at[b],
                         in_sems.at[b])
        pltpu.async_copy(wn_hbm.at[pl.ds(base, CHUNK)], wnv.at[b],
                         in_sems.at[b])

    def _drain(b):
        pltpu.make_async_copy(idx_hbm.at[pl.ds(0, CHUNK)], idxv.at[b],
                              in_sems.at[b]).wait()
        pltpu.make_async_copy(lam16_hbm.at[pl.ds(0, CHUNK)], lamv.at[b],
                              in_sems.at[b]).wait()
        pltpu.make_async_copy(w_hbm.at[pl.ds(0, CHUNK)], wv.at[b],
                              in_sems.at[b]).wait()
        pltpu.make_async_copy(wn_hbm.at[pl.ds(0, CHUNK)], wnv.at[b],
                              in_sems.at[b]).wait()

    _fill(0, 0)
    for k in range(NCHUNK):
        b = k % 2
        if k + 1 < NCHUNK:
            _fill(k + 1, 1 - b)
        _drain(b)
        pltpu.sync_copy(wv.at[b], g_sp.at[idxv.at[b]], add=True)
        pltpu.sync_copy(wnv.at[b], gn_sp.at[idxv.at[b]], add=True)
        pltpu.sync_copy(lamv.at[b], cnt_sp.at[idxv.at[b]], add=True)

    pltpu.sync_copy(g_sp.at[pl.ds(0, 8)], rb)
    pltpu.sync_copy(gn_sp.at[pl.ds(0, 8)], rb)
    pltpu.sync_copy(cnt_sp.at[pl.ds(0, 8)], rb)
    plsc.subcore_barrier()

    @pl.when(sid == 0)
    def _():
        pltpu.sync_copy(g_sp, g_out.at[cid])
        pltpu.sync_copy(gn_sp, gn_out.at[cid])
        pltpu.sync_copy(cnt_sp, cnt_out.at[cid])


def _stage_b_body(xf_ref, lac_ref, lbc_ref, mn_ref, slse_ref, sxx_ref):
    i = pl.program_id(0)

    @pl.when(i == 0)
    def _():
        slse_ref[...] = jnp.zeros_like(slse_ref)
        sxx_ref[...] = jnp.zeros_like(sxx_ref)

    xf = xf_ref[...]
    xssq = jnp.sum(xf * xf, axis=1, keepdims=True)
    xfn = xf * jax.lax.rsqrt(xssq)
    mn = mn_ref[...]
    st = jax.lax.dot_general(xfn, mn, (((1,), (1,)), ((), ())),
                             precision=jax.lax.Precision.DEFAULT,
                             preferred_element_type=jnp.float32)
    z = jnp.sum(jnp.exp(st), axis=1, keepdims=True)
    lse = jnp.log(z - float(CP - C))

    lam2c = lac_ref[...] + lbc_ref[...]
    slse_ref[...] += jnp.sum(lam2c * lse)
    sxx_ref[...] += jnp.sum(lam2c * xssq)


def _stage_c_body(gp_ref, gnp_ref, cntp_ref, h_ref, mnorm2_ref, k_ref,
                  m_ref, ginit_ref, cinit_ref, slse_ref, sxx_ref,
                  loss_ref, gout_ref, cntout_ref):
    gs = gp_ref[0] + gp_ref[1]
    gns = gnp_ref[0] + gnp_ref[1]
    cs = cntp_ref[:CP] + cntp_ref[CP:]
    center = (sxx_ref[0, 0]
              - 2.0 * jnp.sum(gs * m_ref[...])
              + jnp.sum(cs * mnorm2_ref[...])) * (WEIGHT_CENT / B)
    sim = (jnp.sum(cs * h_ref[...])
           + slse_ref[0, 0]
           - jnp.sum(gns * k_ref[...]) * (1.0 / TAU)) * (ETA / B)
    loss_ref[...] = jnp.full((1, 1), sim + center, jnp.float32)
    gout_ref[...] = gs + ginit_ref[...]
    cntout_ref[...] = cs + cinit_ref[...]


@jax.jit
def kernel(xf, target_a, target_b, lam_a, lam_b, epoch, matrix, grad_buf,
           count_buf):
    f32 = jnp.float32
    m_pad = jnp.pad(matrix, ((0, CP - C), (0, 0)))
    g_init = jnp.pad(grad_buf, ((0, CP - C), (0, 0)))
    c_init = jnp.pad(count_buf, ((0, CP - C), (0, 0)))

    lam_cat = jnp.concatenate([lam_a, lam_b]).reshape(2 * B, 1)
    idx_cat = jnp.concatenate([target_a, target_b]).astype(jnp.int32)
    w, wn, lam16 = pl.pallas_call(
        _scale_body,
        grid=(2 * NB,),
        in_specs=[
            pl.BlockSpec((BB, F), lambda i: (i % NB, 0)),
            pl.BlockSpec((BB, 1), lambda i: (i, 0)),
        ],
        out_specs=[pl.BlockSpec((BB, F), lambda i: (i, 0)),
                   pl.BlockSpec((BB, F), lambda i: (i, 0)),
                   pl.BlockSpec((BB, F), lambda i: (i, 0))],
        out_shape=[
            jax.ShapeDtypeStruct((2 * B, F), f32),
            jax.ShapeDtypeStruct((2 * B, F), f32),
            jax.ShapeDtypeStruct((2 * B, F), f32),
        ],
    )(xf, lam_cat)

    zeros_cpf = jnp.zeros((CP, F), f32)
    sc_scatter = pl.kernel(
        _sc_scatter_body,
        out_type=[
            jax.ShapeDtypeStruct((NC, CP, F), f32),
            jax.ShapeDtypeStruct((NC, CP, F), f32),
            jax.ShapeDtypeStruct((NC, CP, F), f32),
        ],
        mesh=plsc.VectorSubcoreMesh(core_axis_name="c", subcore_axis_name="s"),
        scratch_types=[
            pltpu.VMEM_SHARED((CP, F), f32),
            pltpu.VMEM_SHARED((CP, F), f32),
            pltpu.VMEM_SHARED((CP, F), f32),
            pltpu.VMEM((2, CHUNK, F), f32),
            pltpu.VMEM((2, CHUNK, F), f32),
            pltpu.VMEM((2, CHUNK), jnp.int32),
            pltpu.VMEM((2, CHUNK, F), f32),
            pltpu.VMEM((8, F), f32),
            pltpu.SemaphoreType.DMA((2,)),
            pltpu.SemaphoreType.DMA,
        ],
    )
    g_parts, gn_parts, cnt_parts = sc_scatter(
        w, wn, idx_cat, lam16, zeros_cpf)

    mn, k, h, mnorm2 = pl.pallas_call(
        _stage_a_body,
        out_shape=[
            jax.ShapeDtypeStruct((CP, F), f32),
            jax.ShapeDtypeStruct((CP, F), f32),
            jax.ShapeDtypeStruct((CP, 1), f32),
            jax.ShapeDtypeStruct((CP, 1), f32),
        ],
    )(m_pad)

    la_c = lam_a.reshape(B, 1)
    lb_c = lam_b.reshape(B, 1)
    full = lambda r, c: pl.BlockSpec((r, c), lambda i: (0, 0))
    slse, sxx = pl.pallas_call(
        _stage_b_body,
        grid=(NB,),
        in_specs=[
            pl.BlockSpec((BB, F), lambda i: (i, 0)),
            pl.BlockSpec((BB, 1), lambda i: (i, 0)),
            pl.BlockSpec((BB, 1), lambda i: (i, 0)),
            full(CP, F),
        ],
        out_specs=[full(1, 1), full(1, 1)],
        out_shape=[
            jax.ShapeDtypeStruct((1, 1), f32),
            jax.ShapeDtypeStruct((1, 1), f32),
        ],
    )(xf, la_c, lb_c, mn)

    cnt_packed = cnt_parts[:, :, :1].reshape(NC * CP, 1)
    loss, g_out, cnt_out = pl.pallas_call(
        _stage_c_body,
        out_shape=[
            jax.ShapeDtypeStruct((1, 1), f32),
            jax.ShapeDtypeStruct((CP, F), f32),
            jax.ShapeDtypeStruct((CP, 1), f32),
        ],
    )(g_parts, gn_parts, cnt_packed, h, mnorm2, k, m_pad, g_init,
      c_init, slse, sxx)

    loss = jnp.where(epoch >= 0, loss[0, 0], jnp.nan)
    return (loss, g_out[:C], cnt_out[:C])


# speedup vs baseline: 1.2884x; 1.0078x over previous
import functools

import jax
import jax.numpy as jnp
from jax import lax
from jax.experimental import pallas as pl
from jax.experimental.pallas import tpu as pltpu
from jax.experimental.pallas import tpu_sc as plsc

B = 16384
C = 1000
CP = 1024
F = 128
TAU = 2.0
ETA = 0.5
WEIGHT_CENT = 0.003
BB = 1024
NB = B // BB

NC = 2
NS = 16
NW = NC * NS
PER_W = 2 * B // NW
CHUNK = 128
NCHUNK = PER_W // CHUNK

_HIGH = jax.lax.Precision.HIGHEST


def _stage_a_body(m_ref, mn_ref, k_ref, h_ref, mnorm2_ref):
    m = m_ref[...]
    ssq = jnp.sum(m * m, axis=1, keepdims=True)
    inv = jax.lax.rsqrt(jnp.maximum(ssq, 1e-30))
    mn = m * inv
    gwm = jax.lax.dot_general(mn, mn, (((1,), (1,)), ((), ())),
                              precision=_HIGH,
                              preferred_element_type=jnp.float32)
    lg = gwm * (1.0 / TAU)
    e = jnp.exp(lg)
    z = jnp.sum(e, axis=1, keepdims=True) - float(CP - C)
    zinv = 1.0 / z
    logz = jnp.log(z)
    p = e * zinv
    h_all = jnp.sum(p * (lg - logz), axis=1, keepdims=True)
    h_ref[...] = h_all + float(CP - C) * logz * zinv
    k_ref[...] = jax.lax.dot_general(p, mn, (((1,), (0,)), ((), ())),
                                     precision=_HIGH,
                                     preferred_element_type=jnp.float32)
    mn_ref[...] = mn * (1.0 / TAU)
    mnorm2_ref[...] = ssq


def _scale_body(xf_ref, lam_ref, w_ref, wn_ref, lam16_ref):
    xf = xf_ref[...]
    lam = lam_ref[...]
    ssq = jnp.sum(xf * xf, axis=1, keepdims=True)
    w_ref[...] = lam * xf
    wn_ref[...] = (lam * jax.lax.rsqrt(ssq)) * xf
    lane0 = jax.lax.broadcasted_iota(jnp.int32, (1, F), 1) == 0
    lam16_ref[...] = jnp.where(lane0, lam, 0.0)


def _sc_scatter_body(w_hbm, wn_hbm, idx_hbm, lam16_hbm, zeros_hbm,
                     g_out, gn_out, cnt_out,
                     g_sp, gn_sp, cnt_sp,
                     wv, wnv, idxv, lamv, rb, in_sems, out_sem):
    cid = lax.axis_index("c")
    sid = lax.axis_index("s")

    @pl.when(sid == 0)
    def _():
        pltpu.sync_copy(zeros_hbm, g_sp)
        pltpu.sync_copy(zeros_hbm, gn_sp)
        pltpu.sync_copy(zeros_hbm, cnt_sp)

    plsc.subcore_barrier()

    wid = cid * NS + sid
    base_w = wid * PER_W

    def _fill(k, b):
        base = base_w + k * CHUNK
        pltpu.async_copy(idx_hbm.at[pl.ds(base, CHUNK)], idxv.at[b],
                         in_sems.at[b])
        pltpu.async_copy(lam16_hbm.at[pl.ds(base, CHUNK)], lamv.at[b],
                         in_sems.at[b])
        pltpu.async_copy(w_hbm.at[pl.ds(base, CHUNK)], wv.at[b],
                         in_sems.at[b])
        pltpu.async_copy(wn_hbm.at[pl.ds(base, CHUNK)], wnv.at[b],
                         in_sems.at[b])

    def _drain(b):
        pltpu.make_async_copy(idx_hbm.at[pl.ds(0, CHUNK)], idxv.at[b],
                              in_sems.at[b]).wait()
        pltpu.make_async_copy(lam16_hbm.at[pl.ds(0, CHUNK)], lamv.at[b],
                              in_sems.at[b]).wait()
        pltpu.make_async_copy(w_hbm.at[pl.ds(0, CHUNK)], wv.at[b],
                              in_sems.at[b]).wait()
        pltpu.make_async_copy(wn_hbm.at[pl.ds(0, CHUNK)], wnv.at[b],
                              in_sems.at[b]).wait()

    _fill(0, 0)
    for k in range(NCHUNK):
        b = k % 2
        if k + 1 < NCHUNK:
            _fill(k + 1, 1 - b)
        _drain(b)
        pltpu.sync_copy(wv.at[b], g_sp.at[idxv.at[b]], add=True)
        pltpu.sync_copy(wnv.at[b], gn_sp.at[idxv.at[b]], add=True)
        pltpu.sync_copy(lamv.at[b], cnt_sp.at[idxv.at[b]], add=True)

    pltpu.sync_copy(g_sp.at[pl.ds(0, 8)], rb)
    pltpu.sync_copy(gn_sp.at[pl.ds(0, 8)], rb)
    pltpu.sync_copy(cnt_sp.at[pl.ds(0, 8)], rb)
    plsc.subcore_barrier()

    @pl.when(sid == 0)
    def _():
        pltpu.sync_copy(g_sp, g_out.at[cid])
        pltpu.sync_copy(gn_sp, gn_out.at[cid])
        pltpu.sync_copy(cnt_sp, cnt_out.at[cid])


def _stage_b_body(xf_ref, lac_ref, lbc_ref, mn_ref, slse_ref, sxx_ref):
    i = pl.program_id(0)

    @pl.when(i == 0)
    def _():
        slse_ref[...] = jnp.zeros_like(slse_ref)
        sxx_ref[...] = jnp.zeros_like(sxx_ref)

    xf = xf_ref[...]
    xssq = jnp.sum(xf * xf, axis=1, keepdims=True)
    xfn = xf * jax.lax.rsqrt(xssq)
    mn = mn_ref[...]
    st = jax.lax.dot_general(xfn, mn, (((1,), (1,)), ((), ())),
                             precision=jax.lax.Precision.DEFAULT,
                             preferred_element_type=jnp.float32)
    z = jnp.sum(jnp.exp(st), axis=1, keepdims=True)
    lse = jnp.log(z - float(CP - C))

    lam2c = lac_ref[...] + lbc_ref[...]
    slse_ref[...] += jnp.sum(lam2c * lse)
    sxx_ref[...] += jnp.sum(lam2c * xssq)


def _stage_c_body(gp_ref, gnp_ref, cntp_ref, h_ref, mnorm2_ref, k_ref,
                  m_ref, ginit_ref, cinit_ref, slse_ref, sxx_ref,
                  loss_ref, gout_ref, cntout_ref):
    gs = gp_ref[0] + gp_ref[1]
    gns = gnp_ref[0] + gnp_ref[1]
    cs = cntp_ref[:CP] + cntp_ref[CP:]
    center = (sxx_ref[0, 0]
              - 2.0 * jnp.sum(gs * m_ref[...])
              + jnp.sum(cs * mnorm2_ref[...])) * (WEIGHT_CENT / B)
    sim = (jnp.sum(cs * h_ref[...])
           + slse_ref[0, 0]
           - jnp.sum(gns * k_ref[...]) * (1.0 / TAU)) * (ETA / B)
    loss_ref[...] = jnp.full((1, 1), sim + center, jnp.float32)
    gout_ref[...] = gs + ginit_ref[...]
    cntout_ref[...] = cs + cinit_ref[...]


@jax.jit
def kernel(xf, target_a, target_b, lam_a, lam_b, epoch, matrix, grad_buf,
           count_buf):
    f32 = jnp.float32
    m_pad = jnp.pad(matrix, ((0, CP - C), (0, 0)))
    g_init = jnp.pad(grad_buf, ((0, CP - C), (0, 0)))
    c_init = jnp.pad(count_buf, ((0, CP - C), (0, 0)))

    lam_cat = jnp.concatenate([lam_a, lam_b]).reshape(2 * B, 1)
    idx_cat = jnp.concatenate([target_a, target_b]).astype(jnp.int32)
    w, wn, lam16 = pl.pallas_call(
        _scale_body,
        grid=(2 * NB,),
        in_specs=[
            pl.BlockSpec((BB, F), lambda i: (i % NB, 0)),
            pl.BlockSpec((BB, 1), lambda i: (i, 0)),
        ],
        out_specs=[pl.BlockSpec((BB, F), lambda i: (i, 0)),
                   pl.BlockSpec((BB, F), lambda i: (i, 0)),
                   pl.BlockSpec((BB, F), lambda i: (i, 0))],
        out_shape=[
            jax.ShapeDtypeStruct((2 * B, F), f32),
            jax.ShapeDtypeStruct((2 * B, F), f32),
            jax.ShapeDtypeStruct((2 * B, F), f32),
        ],
    )(xf, lam_cat)

    zeros_cpf = jnp.zeros((CP, F), f32)
    sc_scatter = pl.kernel(
        _sc_scatter_body,
        out_type=[
            jax.ShapeDtypeStruct((NC, CP, F), f32),
            jax.ShapeDtypeStruct((NC, CP, F), f32),
            jax.ShapeDtypeStruct((NC, CP, F), f32),
        ],
        mesh=plsc.VectorSubcoreMesh(core_axis_name="c", subcore_axis_name="s"),
        compiler_params=pltpu.CompilerParams(use_tc_tiling_on_sc=True),
        scratch_types=[
            pltpu.VMEM_SHARED((CP, F), f32),
            pltpu.VMEM_SHARED((CP, F), f32),
            pltpu.VMEM_SHARED((CP, F), f32),
            pltpu.VMEM((2, CHUNK, F), f32),
            pltpu.VMEM((2, CHUNK, F), f32),
            pltpu.VMEM((2, CHUNK), jnp.int32),
            pltpu.VMEM((2, CHUNK, F), f32),
            pltpu.VMEM((8, F), f32),
            pltpu.SemaphoreType.DMA((2,)),
            pltpu.SemaphoreType.DMA,
        ],
    )
    g_parts, gn_parts, cnt_parts = sc_scatter(
        w, wn, idx_cat, lam16, zeros_cpf)

    mn, k, h, mnorm2 = pl.pallas_call(
        _stage_a_body,
        out_shape=[
            jax.ShapeDtypeStruct((CP, F), f32),
            jax.ShapeDtypeStruct((CP, F), f32),
            jax.ShapeDtypeStruct((CP, 1), f32),
            jax.ShapeDtypeStruct((CP, 1), f32),
        ],
    )(m_pad)

    la_c = lam_a.reshape(B, 1)
    lb_c = lam_b.reshape(B, 1)
    full = lambda r, c: pl.BlockSpec((r, c), lambda i: (0, 0))
    slse, sxx = pl.pallas_call(
        _stage_b_body,
        grid=(NB,),
        in_specs=[
            pl.BlockSpec((BB, F), lambda i: (i, 0)),
            pl.BlockSpec((BB, 1), lambda i: (i, 0)),
            pl.BlockSpec((BB, 1), lambda i: (i, 0)),
            full(CP, F),
        ],
        out_specs=[full(1, 1), full(1, 1)],
        out_shape=[
            jax.ShapeDtypeStruct((1, 1), f32),
            jax.ShapeDtypeStruct((1, 1), f32),
        ],
    )(xf, la_c, lb_c, mn)

    cnt_packed = cnt_parts[:, :, :1].reshape(NC * CP, 1)
    loss, g_out, cnt_out = pl.pallas_call(
        _stage_c_body,
        out_shape=[
            jax.ShapeDtypeStruct((1, 1), f32),
            jax.ShapeDtypeStruct((CP, F), f32),
            jax.ShapeDtypeStruct((CP, 1), f32),
        ],
    )(g_parts, gn_parts, cnt_packed, h, mnorm2, k, m_pad, g_init,
      c_init, slse, sxx)

    loss = jnp.where(epoch >= 0, loss[0, 0], jnp.nan)
    return (loss, g_out[:C], cnt_out[:C])


# speedup vs baseline: 1.5725x; 1.2205x over previous
import functools

import jax
import jax.numpy as jnp
from jax import lax
from jax.experimental import pallas as pl
from jax.experimental.pallas import tpu as pltpu
from jax.experimental.pallas import tpu_sc as plsc

B = 16384
C = 1000
CP = 1024
F = 128
TAU = 2.0
ETA = 0.5
WEIGHT_CENT = 0.003
BB = 1024
NB = B // BB

NC = 2
NS = 16
NW = NC * NS
PER_W = 2 * B // NW
CHUNK = 128
NCHUNK = PER_W // CHUNK

_HIGH = jax.lax.Precision.HIGHEST


def _stage_a_body(m_ref, tok_ref, mn_ref, k_ref, h_ref, mnorm2_ref):
    del tok_ref
    m = m_ref[...]
    ssq = jnp.sum(m * m, axis=1, keepdims=True)
    inv = jax.lax.rsqrt(jnp.maximum(ssq, 1e-30))
    mn = m * inv
    gwm = jax.lax.dot_general(mn, mn, (((1,), (1,)), ((), ())),
                              precision=jax.lax.Precision.DEFAULT,
                              preferred_element_type=jnp.float32)
    lg = gwm * (1.0 / TAU)
    e = jnp.exp(lg)
    z = jnp.sum(e, axis=1, keepdims=True) - float(CP - C)
    zinv = 1.0 / z
    logz = jnp.log(z)
    p = e * zinv
    h_all = jnp.sum(p * (lg - logz), axis=1, keepdims=True)
    h_ref[...] = h_all + float(CP - C) * logz * zinv
    k_ref[...] = jax.lax.dot_general(p, mn, (((1,), (0,)), ((), ())),
                                     precision=jax.lax.Precision.DEFAULT,
                                     preferred_element_type=jnp.float32)
    mn_ref[...] = mn * (1.0 / TAU)
    mnorm2_ref[...] = ssq


def _scale_body(xf_ref, lam_ref, w_ref, wn_ref, lam16_ref):
    xf = xf_ref[...]
    lam = lam_ref[0].T
    ssq = jnp.sum(xf * xf, axis=1, keepdims=True)
    w_ref[...] = lam * xf
    wn_ref[...] = (lam * jax.lax.rsqrt(ssq)) * xf
    lane0 = jax.lax.broadcasted_iota(jnp.int32, (1, F), 1) == 0
    lam16_ref[...] = jnp.where(lane0, lam, 0.0)


def _sc_scatter_body(w_hbm, wn_hbm, idx_hbm, lam16_hbm, zeros_hbm,
                     g_out, gn_out, cnt_out,
                     g_sp, gn_sp, cnt_sp,
                     wv, wnv, idxv, lamv, rb, in_sems, out_sem):
    cid = lax.axis_index("c")
    sid = lax.axis_index("s")

    @pl.when(sid == 0)
    def _():
        pltpu.sync_copy(zeros_hbm, g_sp)
        pltpu.sync_copy(zeros_hbm, gn_sp)
        pltpu.sync_copy(zeros_hbm, cnt_sp)

    plsc.subcore_barrier()

    wid = cid * NS + sid
    base_w = wid * PER_W

    def _fill(k, b):
        base = base_w + k * CHUNK
        pltpu.async_copy(idx_hbm.at[pl.ds(base, CHUNK)], idxv.at[b],
                         in_sems.at[b])
        pltpu.async_copy(lam16_hbm.at[pl.ds(base, CHUNK)], lamv.at[b],
                         in_sems.at[b])
        pltpu.async_copy(w_hbm.at[pl.ds(base, CHUNK)], wv.at[b],
                         in_sems.at[b])
        pltpu.async_copy(wn_hbm.at[pl.ds(base, CHUNK)], wnv.at[b],
                         in_sems.at[b])

    def _drain(b):
        pltpu.make_async_copy(idx_hbm.at[pl.ds(0, CHUNK)], idxv.at[b],
                              in_sems.at[b]).wait()
        pltpu.make_async_copy(lam16_hbm.at[pl.ds(0, CHUNK)], lamv.at[b],
                              in_sems.at[b]).wait()
        pltpu.make_async_copy(w_hbm.at[pl.ds(0, CHUNK)], wv.at[b],
                              in_sems.at[b]).wait()
        pltpu.make_async_copy(wn_hbm.at[pl.ds(0, CHUNK)], wnv.at[b],
                              in_sems.at[b]).wait()

    _fill(0, 0)
    for k in range(NCHUNK):
        b = k % 2
        if k + 1 < NCHUNK:
            _fill(k + 1, 1 - b)
        _drain(b)
        pltpu.sync_copy(wv.at[b], g_sp.at[idxv.at[b]], add=True)
        pltpu.sync_copy(wnv.at[b], gn_sp.at[idxv.at[b]], add=True)
        pltpu.sync_copy(lamv.at[b], cnt_sp.at[idxv.at[b]], add=True)

    pltpu.sync_copy(g_sp.at[pl.ds(0, 8)], rb)
    pltpu.sync_copy(gn_sp.at[pl.ds(0, 8)], rb)
    pltpu.sync_copy(cnt_sp.at[pl.ds(0, 8)], rb)
    plsc.subcore_barrier()

    @pl.when(sid == 0)
    def _():
        pltpu.sync_copy(g_sp, g_out.at[cid])
        pltpu.sync_copy(gn_sp, gn_out.at[cid])
        pltpu.sync_copy(cnt_sp, cnt_out.at[cid])


def _stage_b_body(xf_ref, lac_ref, lbc_ref, mn_ref, slse_ref, sxx_ref):
    i = pl.program_id(0)

    @pl.when(i == 0)
    def _():
        slse_ref[...] = jnp.zeros_like(slse_ref)
        sxx_ref[...] = jnp.zeros_like(sxx_ref)

    xf = xf_ref[...]
    xssq = jnp.sum(xf * xf, axis=1, keepdims=True)
    xfn = xf * jax.lax.rsqrt(xssq)
    mn = mn_ref[...]
    st = jax.lax.dot_general(xfn, mn, (((1,), (1,)), ((), ())),
                             precision=jax.lax.Precision.DEFAULT,
                             preferred_element_type=jnp.float32)
    z = jnp.sum(jnp.exp(st), axis=1, keepdims=True)
    lse = jnp.log(z - float(CP - C))

    lam2c = (lac_ref[0] + lbc_ref[0]).T
    slse_ref[...] += jnp.sum(lam2c * lse)
    sxx_ref[...] += jnp.sum(lam2c * xssq)


def _stage_c_body(gp_ref, gnp_ref, cntp_ref, h_ref, mnorm2_ref, k_ref,
                  m_ref, ginit_ref, cinit_ref, slse_ref, sxx_ref,
                  loss_ref, gout_ref, cntout_ref):
    gs = gp_ref[0] + gp_ref[1]
    gns = gnp_ref[0] + gnp_ref[1]
    cs = cntp_ref[:CP] + cntp_ref[CP:]
    center = (sxx_ref[0, 0]
              - 2.0 * jnp.sum(gs * m_ref[...])
              + jnp.sum(cs * mnorm2_ref[...])) * (WEIGHT_CENT / B)
    sim = (jnp.sum(cs * h_ref[...])
           + slse_ref[0, 0]
           - jnp.sum(gns * k_ref[...]) * (1.0 / TAU)) * (ETA / B)
    loss_ref[...] = jnp.full((1, 1), sim + center, jnp.float32)
    gout_ref[...] = gs + ginit_ref[...]
    cntout_ref[...] = cs + cinit_ref[...]


@jax.jit
def kernel(xf, target_a, target_b, lam_a, lam_b, epoch, matrix, grad_buf,
           count_buf):
    f32 = jnp.float32
    m_pad = jnp.pad(matrix, ((0, CP - C), (0, 0)))
    g_init = jnp.pad(grad_buf, ((0, CP - C), (0, 0)))
    c_init = jnp.pad(count_buf, ((0, CP - C), (0, 0)))

    lam_cat = jnp.concatenate([lam_a, lam_b]).reshape(2 * NB, 1, BB)
    idx_cat = jnp.concatenate([target_a, target_b]).astype(jnp.int32)
    w, wn, lam16 = pl.pallas_call(
        _scale_body,
        grid=(2 * NB,),
        in_specs=[
            pl.BlockSpec((BB, F), lambda i: (i % NB, 0)),
            pl.BlockSpec((1, 1, BB), lambda i: (i, 0, 0)),
        ],
        out_specs=[pl.BlockSpec((BB, F), lambda i: (i, 0)),
                   pl.BlockSpec((BB, F), lambda i: (i, 0)),
                   pl.BlockSpec((BB, F), lambda i: (i, 0))],
        out_shape=[
            jax.ShapeDtypeStruct((2 * B, F), f32),
            jax.ShapeDtypeStruct((2 * B, F), f32),
            jax.ShapeDtypeStruct((2 * B, F), f32),
        ],
    )(xf, lam_cat)

    zeros_cpf = jnp.zeros((CP, F), f32)
    sc_scatter = pl.kernel(
        _sc_scatter_body,
        out_type=[
            jax.ShapeDtypeStruct((NC, CP, F), f32),
            jax.ShapeDtypeStruct((NC, CP, F), f32),
            jax.ShapeDtypeStruct((NC, CP, F), f32),
        ],
        mesh=plsc.VectorSubcoreMesh(core_axis_name="c", subcore_axis_name="s"),
        compiler_params=pltpu.CompilerParams(use_tc_tiling_on_sc=True),
        scratch_types=[
            pltpu.VMEM_SHARED((CP, F), f32),
            pltpu.VMEM_SHARED((CP, F), f32),
            pltpu.VMEM_SHARED((CP, F), f32),
            pltpu.VMEM((2, CHUNK, F), f32),
            pltpu.VMEM((2, CHUNK, F), f32),
            pltpu.VMEM((2, CHUNK), jnp.int32),
            pltpu.VMEM((2, CHUNK, F), f32),
            pltpu.VMEM((8, F), f32),
            pltpu.SemaphoreType.DMA((2,)),
            pltpu.SemaphoreType.DMA,
        ],
    )
    g_parts, gn_parts, cnt_parts = sc_scatter(
        w, wn, idx_cat, lam16, zeros_cpf)

    mn, k, h, mnorm2 = pl.pallas_call(
        _stage_a_body,
        grid=(1,),
        in_specs=[pl.BlockSpec((CP, F), lambda i: (0, 0)),
                  pl.BlockSpec((8, F), lambda i: (0, 0))],
        out_specs=[pl.BlockSpec((CP, F), lambda i: (0, 0)),
                   pl.BlockSpec((CP, F), lambda i: (0, 0)),
                   pl.BlockSpec((CP, 1), lambda i: (0, 0)),
                   pl.BlockSpec((CP, 1), lambda i: (0, 0))],
        out_shape=[
            jax.ShapeDtypeStruct((CP, F), f32),
            jax.ShapeDtypeStruct((CP, F), f32),
            jax.ShapeDtypeStruct((CP, 1), f32),
            jax.ShapeDtypeStruct((CP, 1), f32),
        ],
    )(m_pad, w)

    la_c = lam_a.reshape(NB, 1, BB)
    lb_c = lam_b.reshape(NB, 1, BB)
    full = lambda r, c: pl.BlockSpec((r, c), lambda i: (0, 0))
    slse, sxx = pl.pallas_call(
        _stage_b_body,
        grid=(NB,),
        in_specs=[
            pl.BlockSpec((BB, F), lambda i: (i, 0)),
            pl.BlockSpec((1, 1, BB), lambda i: (i, 0, 0)),
            pl.BlockSpec((1, 1, BB), lambda i: (i, 0, 0)),
            full(CP, F),
        ],
        out_specs=[full(1, 1), full(1, 1)],
        out_shape=[
            jax.ShapeDtypeStruct((1, 1), f32),
            jax.ShapeDtypeStruct((1, 1), f32),
        ],
    )(xf, la_c, lb_c, mn)

    cnt_packed = cnt_parts[:, :, :1].reshape(NC * CP, 1)
    loss, g_out, cnt_out = pl.pallas_call(
        _stage_c_body,
        out_shape=[
            jax.ShapeDtypeStruct((1, 1), f32),
            jax.ShapeDtypeStruct((CP, F), f32),
            jax.ShapeDtypeStruct((CP, 1), f32),
        ],
    )(g_parts, gn_parts, cnt_packed, h, mnorm2, k, m_pad, g_init,
      c_init, slse, sxx)

    loss = jnp.where(epoch >= 0, loss[0, 0], jnp.nan)
    return (loss, g_out[:C], cnt_out[:C])
